# baseline (speedup 1.0000x reference)
import jax
import jax.numpy as jnp
import numpy as np
from jax import lax
from jax.experimental import pallas as pl
from jax.experimental.pallas import tpu as pltpu
from jax.experimental.pallas import tpu_sc as plsc

NUM_CODE = 8192
CODE_DIM = 256
COMMITMENT_COST = 0.25
NUM_TOK = 8192

BM = 8192
BN = 1024
BC = 512

SC_CORES = 2
SC_SUBCORES = 16
SC_WORKERS = SC_CORES * SC_SUBCORES
TOK_PER_WORKER = NUM_TOK // SC_WORKERS


def _argmax_body(enc_ref, cb_ref, idx_ref, loss_ref, cbq_ref,
                 enc_bf, cb_bf, m_scr, i_scr):
    n = pl.program_id(0)
    t = pl.program_id(1)

    @pl.when((n == 0) & (t == 0))
    def _():
        x = enc_ref[...]
        r = jnp.sqrt(jnp.sum(x * x, axis=1, keepdims=True))
        enc_bf[...] = (x / jnp.maximum(r, 1e-12)).astype(jnp.bfloat16)

    @pl.when(t == 0)
    def _():
        y = cb_ref[...]
        r = jnp.sqrt(jnp.sum(y * y, axis=1, keepdims=True))
        yb = (y / jnp.maximum(r, 1e-12)).astype(jnp.bfloat16)
        cb_bf[...] = yb
        cbq_ref[...] = yb.astype(jnp.float32)

    e = enc_bf[pl.ds(t * BM, BM), :]
    slab_iota = lax.broadcasted_iota(jnp.int32, (8, BM), 0)

    def chunk_dot(j):
        return lax.dot_general(
            cb_bf[pl.ds(j * BC, BC), :], e,
            (((1,), (1,)), ((), ())),
            preferred_element_type=jnp.float32,
        )

    def scan_slabs(s, base, acc):
        rm = lax.slice(s, (0, 0), (8, BM))
        ri = slab_iota + base
        for k in range(1, BC // 8):
            sk = lax.slice(s, (k * 8, 0), ((k + 1) * 8, BM))
            upd = sk > rm
            rm = jnp.where(upd, sk, rm)
            ri = jnp.where(upd, slab_iota + (base + k * 8), ri)
        if acc is None:
            return rm, ri
        acc_m, acc_i = acc
        upd = rm > acc_m
        return jnp.where(upd, rm, acc_m), jnp.where(upd, ri, acc_i)

    n_chunks = BN // BC
    acc = None
    s_prev = chunk_dot(0)
    for j in range(1, n_chunks):
        s_cur = chunk_dot(j)
        acc = scan_slabs(s_prev, n * BN + (j - 1) * BC, acc)
        s_prev = s_cur
    acc_m, acc_i = scan_slabs(s_prev, n * BN + (n_chunks - 1) * BC, acc)

    bm = jnp.max(acc_m, axis=0, keepdims=True)
    bi = jnp.min(jnp.where(acc_m == bm, acc_i, jnp.int32(2**30)),
                 axis=0, keepdims=True)

    row = pl.ds(t, 1)

    @pl.when(n == 0)
    def _():
        m_scr[row, :] = bm
        i_scr[row, :] = bi

    @pl.when(n > 0)
    def _():
        better = bm > m_scr[row, :]
        i_scr[row, :] = jnp.where(better, bi, i_scr[row, :])
        m_scr[row, :] = jnp.maximum(bm, m_scr[row, :])

    @pl.when(n == pl.num_programs(0) - 1)
    def _():
        idx_ref[...] = i_scr[row, :].reshape(1, 1, BM)
        s_total = jnp.sum(m_scr[...])
        denom = float(NUM_TOK) * float(CODE_DIM)
        loss = ((1.0 + COMMITMENT_COST)
                * (2.0 * NUM_TOK - 2.0 * s_total) / denom)
        loss_ref[...] = loss.reshape(1, 1)


def _argmax_similarity(enc, cb):
    grid = (NUM_CODE // BN, NUM_TOK // BM)
    n_t = NUM_TOK // BM
    return pl.pallas_call(
        _argmax_body,
        grid=grid,
        in_specs=[
            pl.BlockSpec((NUM_TOK, CODE_DIM), lambda n, t: (0, 0)),
            pl.BlockSpec((BN, CODE_DIM), lambda n, t: (n, 0)),
        ],
        out_specs=[
            pl.BlockSpec((1, 1, BM), lambda n, t: (t, 0, 0)),
            pl.BlockSpec((1, 1), lambda n, t: (0, 0)),
            pl.BlockSpec((BN, CODE_DIM), lambda n, t: (n, 0)),
        ],
        out_shape=[
            jax.ShapeDtypeStruct((n_t, 1, BM), jnp.int32),
            jax.ShapeDtypeStruct((1, 1), jnp.float32),
            jax.ShapeDtypeStruct((NUM_CODE, CODE_DIM), jnp.float32),
        ],
        scratch_shapes=[
            pltpu.VMEM((NUM_TOK, CODE_DIM), jnp.bfloat16),
            pltpu.VMEM((BN, CODE_DIM), jnp.bfloat16),
            pltpu.VMEM((n_t, BM), jnp.float32),
            pltpu.VMEM((n_t, BM), jnp.int32),
        ],
        compiler_params=pltpu.CompilerParams(
            dimension_semantics=("arbitrary", "arbitrary"),
        ),
    )(enc, cb)


_LUT_SIZE = NUM_TOK + 16
_LUT_P = (np.arange(_LUT_SIZE, dtype=np.float32)
          / np.float32(float(NUM_TOK))).astype(np.float32)
_ENT_LUT = np.repeat(
    (_LUT_P * np.log(_LUT_P + np.float32(1e-10))).astype(np.float32)[:, None],
    128, axis=1)

HIST_PER_SUB = NUM_TOK // SC_SUBCORES
BIN_PER_SUB = NUM_CODE // SC_SUBCORES


def _sc_gather_hist_body(cb_hbm, idx_hbm, lut_hbm, quant_hbm, perp_hbm,
                         idx_g, idx_h, rows_v, ones_v, zeros_v, cnt_v,
                         cnt_i, lut_v, part_v, red_v, out_v, hist_sh,
                         part_sh, sem_g, sem_w):
    c = lax.axis_index("c")
    s = lax.axis_index("s")
    wid = s * SC_CORES + c
    base = wid * TOK_PER_WORKER

    pltpu.sync_copy(idx_hbm.at[pl.ds(base, TOK_PER_WORKER)], idx_g)
    gather = pltpu.async_copy(cb_hbm.at[idx_g], rows_v, sem_g)

    pltpu.sync_copy(idx_hbm.at[pl.ds(s * HIST_PER_SUB, HIST_PER_SUB)], idx_h)
    for i in range(BIN_PER_SUB // 16):
        zeros_v[pl.ds(i * 16, 16)] = jnp.zeros((16,), jnp.float32)
    for i in range(HIST_PER_SUB // 16):
        ones_v[pl.ds(i * 16, 16)] = jnp.ones((16,), jnp.float32)
    pltpu.sync_copy(zeros_v, hist_sh.at[pl.ds(s * BIN_PER_SUB, BIN_PER_SUB)])
    plsc.subcore_barrier()
    pltpu.sync_copy(ones_v, hist_sh.at[idx_h], add=True)

    gather.wait()
    out_cp = pltpu.async_copy(
        rows_v, quant_hbm.at[pl.ds(base, TOK_PER_WORKER)], sem_w)

    plsc.subcore_barrier()

    pltpu.sync_copy(hist_sh.at[pl.ds(s * BIN_PER_SUB, BIN_PER_SUB)], cnt_v)
    for i in range(BIN_PER_SUB // 16):
        cnt_i[pl.ds(i * 16, 16)] = cnt_v[pl.ds(i * 16, 16)].astype(jnp.int32)
    acc = jnp.zeros((16,), jnp.float32)
    for g in range(BIN_PER_SUB // 128):
        pltpu.async_copy(
            lut_hbm.at[cnt_i.at[pl.ds(g * 128, 128)]], lut_v, sem_g).wait()
        for i in range(128):
            acc = acc + lut_v[i, pl.ds(0, 16)]
    part_v[...] = acc
    pltpu.sync_copy(part_v, part_sh.at[pl.ds(s * 16, 16)])
    plsc.subcore_barrier()

    @pl.when((s == 0) & (c == 0))
    def _():
        pltpu.sync_copy(part_sh, red_v)
        tot = jnp.zeros((16,), jnp.float32)
        for i in range(SC_SUBCORES):
            tot = tot + red_v[pl.ds(i * 16, 16)]
        out_v[...] = jnp.exp(-tot)
        pltpu.sync_copy(out_v, perp_hbm)

    out_cp.wait()


def _sc_gather_hist(cb_q, idx, lut):
    return pl.kernel(
        _sc_gather_hist_body,
        out_type=[
            jax.ShapeDtypeStruct((NUM_TOK, CODE_DIM), jnp.float32),
            jax.ShapeDtypeStruct((16,), jnp.float32),
        ],
        mesh=plsc.VectorSubcoreMesh(core_axis_name="c", subcore_axis_name="s"),
        scratch_types=[
            pltpu.VMEM((TOK_PER_WORKER,), jnp.int32),
            pltpu.VMEM((HIST_PER_SUB,), jnp.int32),
            pltpu.VMEM((TOK_PER_WORKER, CODE_DIM), jnp.float32),
            pltpu.VMEM((HIST_PER_SUB,), jnp.float32),
            pltpu.VMEM((BIN_PER_SUB,), jnp.float32),
            pltpu.VMEM((BIN_PER_SUB,), jnp.float32),
            pltpu.VMEM((BIN_PER_SUB,), jnp.int32),
            pltpu.VMEM((128, 128), jnp.float32),
            pltpu.VMEM((16,), jnp.float32),
            pltpu.VMEM((SC_SUBCORES * 16,), jnp.float32),
            pltpu.VMEM((16,), jnp.float32),
            pltpu.VMEM_SHARED((NUM_CODE,), jnp.float32),
            pltpu.VMEM_SHARED((SC_SUBCORES * 16,), jnp.float32),
            pltpu.SemaphoreType.DMA,
            pltpu.SemaphoreType.DMA,
        ],
    )(cb_q, idx, lut)


def kernel(encoded_patch_input, codebook_weight):
    enc = encoded_patch_input.reshape(NUM_TOK, CODE_DIM)
    idx, loss, cb_q = _argmax_similarity(enc, codebook_weight)
    idx_flat = idx.reshape(NUM_TOK)
    lut = jnp.asarray(_ENT_LUT)
    quant, perp_vec = _sc_gather_hist(cb_q, idx_flat, lut)

    B, C, Tn = encoded_patch_input.shape[:3]
    return (
        loss.reshape(()),
        quant.reshape(B, C, Tn, CODE_DIM),
        perp_vec[15].reshape(()),
        codebook_weight,
        idx_flat.reshape(B, C, Tn),
    )


# speedup vs baseline: 3.8995x; 3.8995x over previous
import jax
import jax.numpy as jnp
from jax import lax
from jax.experimental import pallas as pl
from jax.experimental.pallas import tpu as pltpu
from jax.experimental.pallas import tpu_sc as plsc

NUM_CODE = 8192
CODE_DIM = 256
COMMITMENT_COST = 0.25
NUM_TOK = 8192

BM = 8192
BN = 1024
BC = 512

SC_CORES = 2
SC_SUBCORES = 16
SC_WORKERS = SC_CORES * SC_SUBCORES
TOK_PER_WORKER = NUM_TOK // SC_WORKERS


def _argmax_body(enc_ref, cb_ref, idx_ref, sim_ref, cbq_ref,
                 enc_bf, cb_bf, m_scr, i_scr):
    n = pl.program_id(0)
    t = pl.program_id(1)

    @pl.when((n == 0) & (t == 0))
    def _():
        x = enc_ref[...]
        r = jnp.sqrt(jnp.sum(x * x, axis=1, keepdims=True))
        enc_bf[...] = (x / jnp.maximum(r, 1e-12)).astype(jnp.bfloat16)

    @pl.when(t == 0)
    def _():
        y = cb_ref[...]
        r = jnp.sqrt(jnp.sum(y * y, axis=1, keepdims=True))
        yb = (y / jnp.maximum(r, 1e-12)).astype(jnp.bfloat16)
        cb_bf[...] = yb
        cbq_ref[...] = yb.astype(jnp.float32)

    e = enc_bf[pl.ds(t * BM, BM), :]
    slab_iota = lax.broadcasted_iota(jnp.int32, (8, BM), 0)

    def chunk_dot(j):
        return lax.dot_general(
            cb_bf[pl.ds(j * BC, BC), :], e,
            (((1,), (1,)), ((), ())),
            preferred_element_type=jnp.float32,
        )

    def scan_slabs(s, base, acc):
        rm = lax.slice(s, (0, 0), (8, BM))
        ri = slab_iota + base
        for k in range(1, BC // 8):
            sk = lax.slice(s, (k * 8, 0), ((k + 1) * 8, BM))
            upd = sk > rm
            rm = jnp.where(upd, sk, rm)
            ri = jnp.where(upd, slab_iota + (base + k * 8), ri)
        if acc is None:
            return rm, ri
        acc_m, acc_i = acc
        upd = rm > acc_m
        return jnp.where(upd, rm, acc_m), jnp.where(upd, ri, acc_i)

    n_chunks = BN // BC
    acc = None
    s_prev = chunk_dot(0)
    for j in range(1, n_chunks):
        s_cur = chunk_dot(j)
        acc = scan_slabs(s_prev, n * BN + (j - 1) * BC, acc)
        s_prev = s_cur
    acc_m, acc_i = scan_slabs(s_prev, n * BN + (n_chunks - 1) * BC, acc)

    bm = jnp.max(acc_m, axis=0, keepdims=True)
    bi = jnp.min(jnp.where(acc_m == bm, acc_i, jnp.int32(2**30)),
                 axis=0, keepdims=True)

    row = pl.ds(t, 1)

    @pl.when(n == 0)
    def _():
        m_scr[row, :] = bm
        i_scr[row, :] = bi

    @pl.when(n > 0)
    def _():
        better = bm > m_scr[row, :]
        i_scr[row, :] = jnp.where(better, bi, i_scr[row, :])
        m_scr[row, :] = jnp.maximum(bm, m_scr[row, :])

    @pl.when(n == pl.num_programs(0) - 1)
    def _():
        idx_ref[...] = i_scr[row, :].reshape(1, 1, BM)
        sim_ref[...] = m_scr[row, :].reshape(1, 1, BM)


def _argmax_similarity(enc, cb):
    grid = (NUM_CODE // BN, NUM_TOK // BM)
    n_t = NUM_TOK // BM
    return pl.pallas_call(
        _argmax_body,
        grid=grid,
        in_specs=[
            pl.BlockSpec((NUM_TOK, CODE_DIM), lambda n, t: (0, 0)),
            pl.BlockSpec((BN, CODE_DIM), lambda n, t: (n, 0)),
        ],
        out_specs=[
            pl.BlockSpec((1, 1, BM), lambda n, t: (t, 0, 0)),
            pl.BlockSpec((1, 1, BM), lambda n, t: (t, 0, 0)),
            pl.BlockSpec((BN, CODE_DIM), lambda n, t: (n, 0)),
        ],
        out_shape=[
            jax.ShapeDtypeStruct((n_t, 1, BM), jnp.int32),
            jax.ShapeDtypeStruct((n_t, 1, BM), jnp.float32),
            jax.ShapeDtypeStruct((NUM_CODE, CODE_DIM), jnp.float32),
        ],
        scratch_shapes=[
            pltpu.VMEM((NUM_TOK, CODE_DIM), jnp.bfloat16),
            pltpu.VMEM((BN, CODE_DIM), jnp.bfloat16),
            pltpu.VMEM((n_t, BM), jnp.float32),
            pltpu.VMEM((n_t, BM), jnp.int32),
        ],
        compiler_params=pltpu.CompilerParams(
            dimension_semantics=("arbitrary", "arbitrary"),
        ),
    )(enc, cb)


def _sc_gather_hist_body(cb_hbm, idx_hbm, quant_hbm, counts_hbm,
                         idx_v, rows_v, ones_v, zeros_v, hist_sh,
                         sem_g, sem_w):
    c = lax.axis_index("c")
    s = lax.axis_index("s")
    wid = s * SC_CORES + c
    base = wid * TOK_PER_WORKER
    z_per_sub = NUM_CODE // SC_SUBCORES

    pltpu.sync_copy(idx_hbm.at[pl.ds(base, TOK_PER_WORKER)], idx_v)
    gather = pltpu.async_copy(cb_hbm.at[idx_v], rows_v, sem_g)

    for i in range(z_per_sub // 16):
        zeros_v[pl.ds(i * 16, 16)] = jnp.zeros((16,), jnp.float32)
    for i in range(TOK_PER_WORKER // 16):
        ones_v[pl.ds(i * 16, 16)] = jnp.ones((16,), jnp.float32)
    pltpu.sync_copy(zeros_v, hist_sh.at[pl.ds(s * z_per_sub, z_per_sub)])
    plsc.subcore_barrier()
    pltpu.sync_copy(ones_v, hist_sh.at[idx_v], add=True)

    gather.wait()
    out_cp = pltpu.async_copy(
        rows_v, quant_hbm.at[pl.ds(base, TOK_PER_WORKER)], sem_w)

    plsc.subcore_barrier()

    @pl.when(s == 0)
    def _():
        pltpu.sync_copy(hist_sh, counts_hbm.at[c])

    out_cp.wait()


def _sc_gather_hist(cb_q, idx):
    return pl.kernel(
        _sc_gather_hist_body,
        out_type=[
            jax.ShapeDtypeStruct((NUM_TOK, CODE_DIM), jnp.float32),
            jax.ShapeDtypeStruct((SC_CORES, NUM_CODE), jnp.float32),
        ],
        mesh=plsc.VectorSubcoreMesh(core_axis_name="c", subcore_axis_name="s"),
        scratch_types=[
            pltpu.VMEM((TOK_PER_WORKER,), jnp.int32),
            pltpu.VMEM((TOK_PER_WORKER, CODE_DIM), jnp.float32),
            pltpu.VMEM((TOK_PER_WORKER,), jnp.float32),
            pltpu.VMEM((NUM_CODE // SC_SUBCORES,), jnp.float32),
            pltpu.VMEM_SHARED((NUM_CODE,), jnp.float32),
            pltpu.SemaphoreType.DMA,
            pltpu.SemaphoreType.DMA,
        ],
    )(cb_q, idx)


def _finalize_body(sim_ref, cnt_ref, loss_ref, perp_ref):
    s_sum = jnp.sum(sim_ref[...])
    denom = float(NUM_TOK) * float(CODE_DIM)
    loss = (1.0 + COMMITMENT_COST) * (2.0 * NUM_TOK - 2.0 * s_sum) / denom
    loss_ref[...] = loss.reshape(1, 1)

    cnt = cnt_ref[...]
    p = (cnt[0:1, :] + cnt[1:2, :]) * (1.0 / NUM_TOK)
    ent = -jnp.sum(p * jnp.log(p + 1e-10))
    perp_ref[...] = jnp.exp(ent).reshape(1, 1)


def _finalize(sim, counts):
    return pl.pallas_call(
        _finalize_body,
        out_shape=[
            jax.ShapeDtypeStruct((1, 1), jnp.float32),
            jax.ShapeDtypeStruct((1, 1), jnp.float32),
        ],
    )(sim, counts)


def kernel(encoded_patch_input, codebook_weight):
    enc = encoded_patch_input.reshape(NUM_TOK, CODE_DIM)
    idx, sim, cb_q = _argmax_similarity(enc, codebook_weight)
    idx_flat = idx.reshape(NUM_TOK)
    quant, counts = _sc_gather_hist(cb_q, idx_flat)
    loss, perp = _finalize(sim.reshape(64, 128), counts)

    B, C, Tn = encoded_patch_input.shape[:3]
    return (
        loss.reshape(()),
        quant.reshape(B, C, Tn, CODE_DIM),
        perp.reshape(()),
        codebook_weight,
        idx_flat.reshape(B, C, Tn),
    )


# speedup vs baseline: 3.9946x; 1.0244x over previous
import jax
import jax.numpy as jnp
from jax import lax
from jax.experimental import pallas as pl
from jax.experimental.pallas import tpu as pltpu
from jax.experimental.pallas import tpu_sc as plsc

NUM_CODE = 8192
CODE_DIM = 256
COMMITMENT_COST = 0.25
NUM_TOK = 8192

BM = 8192
BN = 2048
BC = 512

SC_CORES = 2
SC_SUBCORES = 16
SC_WORKERS = SC_CORES * SC_SUBCORES
TOK_PER_WORKER = NUM_TOK // SC_WORKERS


def _argmax_body(enc_ref, cb_ref, idx_ref, sim_ref, cbq_ref,
                 enc_bf, cb_bf, m_scr, i_scr):
    n = pl.program_id(0)
    t = pl.program_id(1)

    @pl.when((n == 0) & (t == 0))
    def _():
        x = enc_ref[...]
        r = jnp.sqrt(jnp.sum(x * x, axis=1, keepdims=True))
        enc_bf[...] = (x / jnp.maximum(r, 1e-12)).astype(jnp.bfloat16)

    @pl.when(t == 0)
    def _():
        y = cb_ref[...]
        r = jnp.sqrt(jnp.sum(y * y, axis=1, keepdims=True))
        yb = (y / jnp.maximum(r, 1e-12)).astype(jnp.bfloat16)
        cb_bf[...] = yb
        cbq_ref[...] = yb.astype(jnp.float32)

    e = enc_bf[pl.ds(t * BM, BM), :]
    slab_iota = lax.broadcasted_iota(jnp.int32, (8, BM), 0)

    def chunk_dot(j):
        return lax.dot_general(
            cb_bf[pl.ds(j * BC, BC), :], e,
            (((1,), (1,)), ((), ())),
            preferred_element_type=jnp.float32,
        )

    def scan_slabs(s, base, acc):
        rm = lax.slice(s, (0, 0), (8, BM))
        ri = slab_iota + base
        for k in range(1, BC // 8):
            sk = lax.slice(s, (k * 8, 0), ((k + 1) * 8, BM))
            upd = sk > rm
            rm = jnp.where(upd, sk, rm)
            ri = jnp.where(upd, slab_iota + (base + k * 8), ri)
        if acc is None:
            return rm, ri
        acc_m, acc_i = acc
        upd = rm > acc_m
        return jnp.where(upd, rm, acc_m), jnp.where(upd, ri, acc_i)

    n_chunks = BN // BC
    acc = None
    s_prev = chunk_dot(0)
    for j in range(1, n_chunks):
        s_cur = chunk_dot(j)
        acc = scan_slabs(s_prev, n * BN + (j - 1) * BC, acc)
        s_prev = s_cur
    acc_m, acc_i = scan_slabs(s_prev, n * BN + (n_chunks - 1) * BC, acc)

    bm = jnp.max(acc_m, axis=0, keepdims=True)
    bi = jnp.min(jnp.where(acc_m == bm, acc_i, jnp.int32(2**30)),
                 axis=0, keepdims=True)

    row = pl.ds(t, 1)

    @pl.when(n == 0)
    def _():
        m_scr[row, :] = bm
        i_scr[row, :] = bi

    @pl.when(n > 0)
    def _():
        better = bm > m_scr[row, :]
        i_scr[row, :] = jnp.where(better, bi, i_scr[row, :])
        m_scr[row, :] = jnp.maximum(bm, m_scr[row, :])

    @pl.when(n == pl.num_programs(0) - 1)
    def _():
        idx_ref[...] = i_scr[row, :].reshape(1, 1, BM)
        sim_ref[...] = m_scr[row, :].reshape(1, 1, BM)


def _argmax_similarity(enc, cb):
    grid = (NUM_CODE // BN, NUM_TOK // BM)
    n_t = NUM_TOK // BM
    return pl.pallas_call(
        _argmax_body,
        grid=grid,
        in_specs=[
            pl.BlockSpec((NUM_TOK, CODE_DIM), lambda n, t: (0, 0)),
            pl.BlockSpec((BN, CODE_DIM), lambda n, t: (n, 0)),
        ],
        out_specs=[
            pl.BlockSpec((1, 1, BM), lambda n, t: (t, 0, 0)),
            pl.BlockSpec((1, 1, BM), lambda n, t: (t, 0, 0)),
            pl.BlockSpec((BN, CODE_DIM), lambda n, t: (n, 0)),
        ],
        out_shape=[
            jax.ShapeDtypeStruct((n_t, 1, BM), jnp.int32),
            jax.ShapeDtypeStruct((n_t, 1, BM), jnp.float32),
            jax.ShapeDtypeStruct((NUM_CODE, CODE_DIM), jnp.float32),
        ],
        scratch_shapes=[
            pltpu.VMEM((NUM_TOK, CODE_DIM), jnp.bfloat16),
            pltpu.VMEM((BN, CODE_DIM), jnp.bfloat16),
            pltpu.VMEM((n_t, BM), jnp.float32),
            pltpu.VMEM((n_t, BM), jnp.int32),
        ],
        compiler_params=pltpu.CompilerParams(
            dimension_semantics=("arbitrary", "arbitrary"),
        ),
    )(enc, cb)


def _sc_gather_hist_body(cb_hbm, idx_hbm, quant_hbm, counts_hbm,
                         idx_v, rows_v, ones_v, zeros_v, hist_sh,
                         sem_g, sem_w):
    c = lax.axis_index("c")
    s = lax.axis_index("s")
    wid = s * SC_CORES + c
    base = wid * TOK_PER_WORKER
    z_per_sub = NUM_CODE // SC_SUBCORES

    pltpu.sync_copy(idx_hbm.at[pl.ds(base, TOK_PER_WORKER)], idx_v)
    gather = pltpu.async_copy(cb_hbm.at[idx_v], rows_v, sem_g)

    for i in range(z_per_sub // 16):
        zeros_v[pl.ds(i * 16, 16)] = jnp.zeros((16,), jnp.float32)
    for i in range(TOK_PER_WORKER // 16):
        ones_v[pl.ds(i * 16, 16)] = jnp.ones((16,), jnp.float32)
    pltpu.sync_copy(zeros_v, hist_sh.at[pl.ds(s * z_per_sub, z_per_sub)])
    plsc.subcore_barrier()
    pltpu.sync_copy(ones_v, hist_sh.at[idx_v], add=True)

    gather.wait()
    out_cp = pltpu.async_copy(
        rows_v, quant_hbm.at[pl.ds(base, TOK_PER_WORKER)], sem_w)

    plsc.subcore_barrier()

    @pl.when(s == 0)
    def _():
        pltpu.sync_copy(hist_sh, counts_hbm.at[c])

    out_cp.wait()


def _sc_gather_hist(cb_q, idx):
    return pl.kernel(
        _sc_gather_hist_body,
        out_type=[
            jax.ShapeDtypeStruct((NUM_TOK, CODE_DIM), jnp.float32),
            jax.ShapeDtypeStruct((SC_CORES, NUM_CODE), jnp.float32),
        ],
        mesh=plsc.VectorSubcoreMesh(core_axis_name="c", subcore_axis_name="s"),
        scratch_types=[
            pltpu.VMEM((TOK_PER_WORKER,), jnp.int32),
            pltpu.VMEM((TOK_PER_WORKER, CODE_DIM), jnp.float32),
            pltpu.VMEM((TOK_PER_WORKER,), jnp.float32),
            pltpu.VMEM((NUM_CODE // SC_SUBCORES,), jnp.float32),
            pltpu.VMEM_SHARED((NUM_CODE,), jnp.float32),
            pltpu.SemaphoreType.DMA,
            pltpu.SemaphoreType.DMA,
        ],
    )(cb_q, idx)


def _finalize_body(sim_ref, cnt_ref, loss_ref, perp_ref):
    s_sum = jnp.sum(sim_ref[...])
    denom = float(NUM_TOK) * float(CODE_DIM)
    loss = (1.0 + COMMITMENT_COST) * (2.0 * NUM_TOK - 2.0 * s_sum) / denom
    loss_ref[...] = loss.reshape(1, 1)

    cnt = cnt_ref[...]
    p = (cnt[0:1, :] + cnt[1:2, :]) * (1.0 / NUM_TOK)
    ent = -jnp.sum(p * jnp.log(p + 1e-10))
    perp_ref[...] = jnp.exp(ent).reshape(1, 1)


def _finalize(sim, counts):
    return pl.pallas_call(
        _finalize_body,
        out_shape=[
            jax.ShapeDtypeStruct((1, 1), jnp.float32),
            jax.ShapeDtypeStruct((1, 1), jnp.float32),
        ],
    )(sim, counts)


def kernel(encoded_patch_input, codebook_weight):
    enc = encoded_patch_input.reshape(NUM_TOK, CODE_DIM)
    idx, sim, cb_q = _argmax_similarity(enc, codebook_weight)
    idx_flat = idx.reshape(NUM_TOK)
    quant, counts = _sc_gather_hist(cb_q, idx_flat)
    loss, perp = _finalize(sim.reshape(64, 128), counts)

    B, C, Tn = encoded_patch_input.shape[:3]
    return (
        loss.reshape(()),
        quant.reshape(B, C, Tn, CODE_DIM),
        perp.reshape(()),
        codebook_weight,
        idx_flat.reshape(B, C, Tn),
    )
